# baseline (speedup 1.0000x reference)
import functools

import jax
import jax.numpy as jnp
from jax import lax
from jax.experimental import pallas as pl
from jax.experimental.pallas import tpu as pltpu
from jax.experimental.pallas import tpu_sc as plsc

VOCAB = 1000000
EMBED = 32
BATCH = 4096
SEQ = 200

_info = plsc.get_sparse_core_info()
NC = _info.num_cores
NS = _info.num_subcores
NW = NC * NS

SB = SEQ // 8
CB = BATCH // 128
FB = EMBED // 8
TI = 128
L = 16
SG = 4
NR = SEQ // SG


NFULL = VOCAB // TI
NTAIL = VOCAB - NFULL * TI


def _sc_convert(t_t, tail):
    mesh = plsc.VectorSubcoreMesh(core_axis_name="c", subcore_axis_name="s")

    GRP = 2
    CW = GRP * TI
    ORW = GRP * EMBED
    NGT = NFULL // GRP
    NRF = NGT // NW

    @functools.partial(
        pl.kernel,
        mesh=mesh,
        out_type=jax.ShapeDtypeStruct((VOCAB * EMBED // TI, TI), jnp.float32),
        scratch_types=[
            pltpu.VMEM((2, FB, GRP, 8, TI), jnp.float32),
            pltpu.VMEM((2, ORW, TI), jnp.float32),
            pltpu.VMEM((NTAIL * EMBED // TI, TI), jnp.float32),
            pltpu.SemaphoreType.DMA((2,)),
            pltpu.SemaphoreType.DMA((2,)),
            pltpu.SemaphoreType.DMA,
        ],
        compiler_params=pltpu.CompilerParams(
            use_tc_tiling_on_sc=True, needs_layout_passes=False),
    )
    def k(tt_hbm, tail_hbm, out_hbm, vbuf, obuf, tbuf, gsem, wsem, tsem):
        w = lax.axis_index("s") * NC + lax.axis_index("c")
        z = w * 0
        has_extra = w < NGT - NRF * NW

        def in_start(r, b):
            g = r * NW + w
            for fb in range(FB):
                for cb2 in range(GRP):
                    pltpu.async_copy(
                        tt_hbm.at[pl.ds(fb * 8, 8),
                                  pl.ds((g * GRP + cb2) * TI, TI)],
                        vbuf.at[b, fb, cb2], gsem.at[b])

        def in_wait(r, b):
            g = r * NW + w
            for fb in range(FB):
                for cb2 in range(GRP):
                    pltpu.make_async_copy(
                        tt_hbm.at[pl.ds(fb * 8, 8),
                                  pl.ds((g * GRP + cb2) * TI, TI)],
                        vbuf.at[b, fb, cb2], gsem.at[b]).wait()

        def out_start(r, b):
            g = r * NW + w
            pltpu.async_copy(
                obuf.at[b], out_hbm.at[pl.ds(g * ORW, ORW)], wsem.at[b])

        def out_wait(r, b):
            g = r * NW + w
            pltpu.make_async_copy(
                obuf.at[b], out_hbm.at[pl.ds(g * ORW, ORW)], wsem.at[b]).wait()

        def transpose_tiles(b):
            vb = vbuf.at[b]
            fbv = [((c0 % EMBED) + lax.iota(jnp.int32, L)) // 8
                   for c0 in (0, 16)]
            fiv = [((c0 % EMBED) + lax.iota(jnp.int32, L)) % 8
                   for c0 in (0, 16)]
            cbv = [jnp.full((L,), cb2, jnp.int32) for cb2 in range(GRP)]

            def tbody(r4, carry):
                for cb2 in range(GRP):
                    for c0g in range(TI // L):
                        t = r4 * 4 + c0g // 2
                        tv = jnp.full((L,), t + z, jnp.int32)
                        vals = plsc.load_gather(
                            vb, [fbv[c0g % 2], cbv[cb2], fiv[c0g % 2], tv])
                        obuf[b, cb2 * EMBED + r4, pl.ds(c0g * L, L)] = vals
                return carry

            lax.fori_loop(0, EMBED, tbody, 0)

        in_start(0, 0)

        def body(g, carry):
            for b in range(2):
                r = g * 2 + b
                bp = 1 - b

                @pl.when(jnp.logical_or(r + 1 < NRF,
                                        jnp.logical_and(r + 1 == NRF, has_extra)))
                def _():
                    in_start(r + 1, bp)

                in_wait(r, b)

                @pl.when(r >= 2)
                def _():
                    out_wait(r - 2, b)

                transpose_tiles(b)
                out_start(r, b)
            return carry

        lax.fori_loop(0, NRF // 2, body, 0)

        out_wait(NRF - 2, 0)

        @pl.when(has_extra)
        def _():
            in_wait(NRF, 0)
            transpose_tiles(0)
            out_start(NRF, 0)
            out_wait(NRF, 0)

        out_wait(NRF - 1, 1)

        @pl.when(w == 0)
        def _():
            pltpu.async_copy(tail_hbm, tbuf, tsem).wait()
            pltpu.sync_copy(tbuf, out_hbm.at[pl.ds(NFULL * EMBED, NTAIL * EMBED // TI)])

    return k(t_t, tail)


def _sc_gather(idxp, table):
    mesh = plsc.VectorSubcoreMesh(core_axis_name="c", subcore_axis_name="s")

    @functools.partial(
        pl.kernel,
        mesh=mesh,
        out_type=jax.ShapeDtypeStruct((SEQ, FB, CB, 8, TI), jnp.float32),
        scratch_types=[
            pltpu.VMEM((2, SG, TI), jnp.int32),
            pltpu.VMEM((2, SG, TI, EMBED), jnp.float32),
            pltpu.VMEM((2, SG, FB, 8, TI), jnp.float32),
            pltpu.SemaphoreType.DMA((2,)),
            pltpu.SemaphoreType.DMA((2,)),
            pltpu.SemaphoreType.DMA((2,)),
        ],
        compiler_params=pltpu.CompilerParams(
            use_tc_tiling_on_sc=False, needs_layout_passes=False),
    )
    def k(idx_hbm, table_hbm, out_hbm, idx_v, rows_v, obuf, isem, gsem, wsem):
        w = lax.axis_index("s") * NC + lax.axis_index("c")
        z = w * 0

        def idx_load(r, b):
            pltpu.async_copy(
                idx_hbm.at[r // 2, w, pl.ds((r % 2) * SG, SG)],
                idx_v.at[b], isem.at[b])

        def idx_wait(r, b):
            pltpu.make_async_copy(
                idx_hbm.at[r // 2, w, pl.ds((r % 2) * SG, SG)],
                idx_v.at[b], isem.at[b]).wait()

        def gathers_start(b):
            for si in range(SG):
                pltpu.async_copy(
                    table_hbm.at[idx_v.at[b, si]], rows_v.at[b, si],
                    gsem.at[b])

        def gathers_wait(b):
            for si in range(SG):
                pltpu.make_async_copy(
                    table_hbm.at[idx_v.at[b, si]], rows_v.at[b, si],
                    gsem.at[b]).wait()

        def wb_start(r, b):
            pltpu.async_copy(
                obuf.at[b], out_hbm.at[pl.ds(r * SG, SG), :, w], wsem.at[b])

        def wb_wait(r, b):
            pltpu.make_async_copy(
                obuf.at[b], out_hbm.at[pl.ds(r * SG, SG), :, w],
                wsem.at[b]).wait()

        def transpose_round(b):
            base = lax.iota(jnp.int32, L)

            def tbody(t0, carry):
                rowv = base + t0 * L
                for si in range(SG):
                    rows = rows_v.at[b, si]
                    for f in range(EMBED):
                        fv = jnp.full((L,), f + z, jnp.int32)
                        obuf[b, si, f // 8, f % 8, pl.ds(t0 * L, L)] = (
                            plsc.load_gather(rows, [rowv, fv]))
                return carry

            lax.fori_loop(0, TI // L, tbody, 0)

        idx_load(0, 0)
        idx_load(1, 1)

        def body(g, carry):
            for b in range(2):
                r = g * 2 + b
                bp = 1 - b
                idx_wait(r, b)
                gathers_start(b)

                @pl.when(r >= 1)
                def _():
                    gathers_wait(bp)

                    @pl.when(r + 1 < NR)
                    def _():
                        idx_load(r + 1, bp)

                    @pl.when(r >= 3)
                    def _():
                        wb_wait(r - 3, bp)

                    transpose_round(bp)
                    wb_start(r - 1, bp)
            return carry

        lax.fori_loop(0, NR // 2, body, 0)

        gathers_wait(1)
        wb_wait(NR - 3, 1)
        transpose_round(1)
        wb_start(NR - 1, 1)
        wb_wait(NR - 2, 0)
        wb_wait(NR - 1, 1)

    return k(idxp, table)


def kernel(token_ids, table):
    tail = table[NFULL * TI:, :].reshape(NTAIL * EMBED // TI, TI)
    conv = _sc_convert(table.T, tail)
    tab = conv.reshape(VOCAB, EMBED)
    idxp = token_ids.T.reshape(SB, 8, CB, TI).transpose(0, 2, 1, 3)
    o = _sc_gather(idxp, tab)
    return o.transpose(2, 4, 0, 1, 3).reshape(BATCH, SEQ, EMBED)


# speedup vs baseline: 1.2974x; 1.2974x over previous
import functools

import jax
import jax.numpy as jnp
from jax import lax
from jax.experimental import pallas as pl
from jax.experimental.pallas import tpu as pltpu
from jax.experimental.pallas import tpu_sc as plsc

VOCAB = 1000000
EMBED = 32
BATCH = 4096
SEQ = 200

_info = plsc.get_sparse_core_info()
NC = _info.num_cores
NS = _info.num_subcores
NW = NC * NS

SB = SEQ // 8
CB = BATCH // 128
FB = EMBED // 8
TI = 128
L = 16
SG = 2
NR = SEQ // SG


NFULL = VOCAB // TI
NTAIL = VOCAB - NFULL * TI


def _sc_convert(t_t, tail):
    mesh = plsc.VectorSubcoreMesh(core_axis_name="c", subcore_axis_name="s")

    GRP = 1
    CW = GRP * TI
    ORW = GRP * EMBED
    NGT = NFULL // GRP
    NRF = NGT // NW

    @functools.partial(
        pl.kernel,
        mesh=mesh,
        out_type=jax.ShapeDtypeStruct((VOCAB * EMBED // TI, TI), jnp.float32),
        scratch_types=[
            pltpu.VMEM((2, FB * GRP * 8, TI), jnp.float32),
            pltpu.VMEM((2, ORW, TI), jnp.float32),
            pltpu.VMEM((NTAIL * EMBED // TI, TI), jnp.float32),
            pltpu.SemaphoreType.DMA((2,)),
            pltpu.SemaphoreType.DMA((2,)),
            pltpu.SemaphoreType.DMA,
        ],
        compiler_params=pltpu.CompilerParams(
            use_tc_tiling_on_sc=True, needs_layout_passes=False),
    )
    def k(tt_hbm, tail_hbm, out_hbm, vbuf, obuf, tbuf, gsem, wsem, tsem):
        w = lax.axis_index("s") * NC + lax.axis_index("c")
        z = w * 0
        has_extra = w < NGT - NRF * NW

        def in_start(r, b):
            g = r * NW + w
            for fb in range(FB):
                for cb2 in range(GRP):
                    pltpu.async_copy(
                        tt_hbm.at[pl.ds(fb * 8, 8),
                                  pl.ds((g * GRP + cb2) * TI, TI)],
                        vbuf.at[b, pl.ds((fb * GRP + cb2) * 8, 8)],
                        gsem.at[b])

        def in_wait(r, b):
            g = r * NW + w
            for fb in range(FB):
                for cb2 in range(GRP):
                    pltpu.make_async_copy(
                        tt_hbm.at[pl.ds(fb * 8, 8),
                                  pl.ds((g * GRP + cb2) * TI, TI)],
                        vbuf.at[b, pl.ds((fb * GRP + cb2) * 8, 8)],
                        gsem.at[b]).wait()

        def out_start(r, b):
            g = r * NW + w
            pltpu.async_copy(
                obuf.at[b], out_hbm.at[pl.ds(g * ORW, ORW)], wsem.at[b])

        def out_wait(r, b):
            g = r * NW + w
            pltpu.make_async_copy(
                obuf.at[b], out_hbm.at[pl.ds(g * ORW, ORW)], wsem.at[b]).wait()

        def transpose_tiles(b):
            vb = vbuf.at[b]
            fl = lax.iota(jnp.int32, L)
            rowc = [[(((c00 + fl) // 8) * GRP + cb2) * 8 + (c00 + fl) % 8
                     for cb2 in range(GRP)] for c00 in (0, 16)]

            for cb2 in range(GRP):
                for r4 in range(EMBED):
                    tvs = [jnp.full((L,), r4 * 4 + q + z, jnp.int32)
                           for q in range(4)]
                    vals = [plsc.load_gather(
                        vb, [rowc[c0g % 2][cb2], tvs[c0g // 2]])
                        for c0g in range(TI // L)]
                    for c0g in range(TI // L):
                        obuf[b, cb2 * EMBED + r4, pl.ds(c0g * L, L)] = vals[c0g]

        in_start(0, 0)

        def body(g, carry):
            for b in range(2):
                r = g * 2 + b
                bp = 1 - b

                @pl.when(jnp.logical_or(r + 1 < NRF,
                                        jnp.logical_and(r + 1 == NRF, has_extra)))
                def _():
                    in_start(r + 1, bp)

                in_wait(r, b)

                @pl.when(r >= 2)
                def _():
                    out_wait(r - 2, b)

                transpose_tiles(b)
                out_start(r, b)
            return carry

        lax.fori_loop(0, NRF // 2, body, 0)

        out_wait(NRF - 2, 0)

        @pl.when(has_extra)
        def _():
            in_wait(NRF, 0)
            transpose_tiles(0)
            out_start(NRF, 0)
            out_wait(NRF, 0)

        out_wait(NRF - 1, 1)

        @pl.when(w == 0)
        def _():
            pltpu.async_copy(tail_hbm, tbuf, tsem).wait()
            pltpu.sync_copy(tbuf, out_hbm.at[pl.ds(NFULL * EMBED, NTAIL * EMBED // TI)])

    return k(t_t, tail)


def _sc_gather(idxp, table):
    mesh = plsc.VectorSubcoreMesh(core_axis_name="c", subcore_axis_name="s")

    @functools.partial(
        pl.kernel,
        mesh=mesh,
        out_type=jax.ShapeDtypeStruct((SEQ, FB, CB, 8, TI), jnp.float32),
        scratch_types=[
            pltpu.VMEM((2, SG, TI), jnp.int32),
            pltpu.VMEM((2, SG, TI, EMBED), jnp.float32),
            pltpu.VMEM((2, SG, FB, 8, TI), jnp.float32),
            pltpu.SemaphoreType.DMA((2,)),
            pltpu.SemaphoreType.DMA((2,)),
            pltpu.SemaphoreType.DMA((2,)),
        ],
        compiler_params=pltpu.CompilerParams(
            use_tc_tiling_on_sc=False, needs_layout_passes=False),
    )
    def k(idx_hbm, table_hbm, out_hbm, idx_v, rows_v, obuf, isem, gsem, wsem):
        w = lax.axis_index("s") * NC + lax.axis_index("c")
        z = w * 0

        PR = 8 // SG

        def idx_load(r, b):
            pltpu.async_copy(
                idx_hbm.at[r // PR, w, pl.ds((r % PR) * SG, SG)],
                idx_v.at[b], isem.at[b])

        def idx_wait(r, b):
            pltpu.make_async_copy(
                idx_hbm.at[r // PR, w, pl.ds((r % PR) * SG, SG)],
                idx_v.at[b], isem.at[b]).wait()

        def gathers_start(b):
            for si in range(SG):
                pltpu.async_copy(
                    table_hbm.at[idx_v.at[b, si]], rows_v.at[b, si],
                    gsem.at[b])

        def gathers_wait(b):
            for si in range(SG):
                pltpu.make_async_copy(
                    table_hbm.at[idx_v.at[b, si]], rows_v.at[b, si],
                    gsem.at[b]).wait()

        def wb_start(r, b):
            pltpu.async_copy(
                obuf.at[b], out_hbm.at[pl.ds(r * SG, SG), :, w], wsem.at[b])

        def wb_wait(r, b):
            pltpu.make_async_copy(
                obuf.at[b], out_hbm.at[pl.ds(r * SG, SG), :, w],
                wsem.at[b]).wait()

        def transpose_round(b):
            rowv = [lax.iota(jnp.int32, L) + t0 * L for t0 in range(TI // L)]
            for si in range(SG):
                rows = rows_v.at[b, si]
                for f in range(EMBED):
                    fv = jnp.full((L,), f + z, jnp.int32)
                    vals = [plsc.load_gather(rows, [rowv[t0], fv])
                            for t0 in range(TI // L)]
                    for t0 in range(TI // L):
                        obuf[b, si, f // 8, f % 8, pl.ds(t0 * L, L)] = vals[t0]

        idx_load(0, 0)
        idx_load(1, 1)

        def body(g, carry):
            for b in range(2):
                r = g * 2 + b
                bp = 1 - b
                idx_wait(r, b)
                gathers_start(b)

                @pl.when(r >= 1)
                def _():
                    gathers_wait(bp)

                    @pl.when(r + 1 < NR)
                    def _():
                        idx_load(r + 1, bp)

                    @pl.when(r >= 3)
                    def _():
                        wb_wait(r - 3, bp)

                    transpose_round(bp)
                    wb_start(r - 1, bp)
            return carry

        lax.fori_loop(0, NR // 2, body, 0)

        gathers_wait(1)
        wb_wait(NR - 3, 1)
        transpose_round(1)
        wb_start(NR - 1, 1)
        wb_wait(NR - 2, 0)
        wb_wait(NR - 1, 1)

    return k(idxp, table)


def kernel(token_ids, table):
    tail = table[NFULL * TI:, :].reshape(NTAIL * EMBED // TI, TI)
    conv = _sc_convert(table.T, tail)
    tab = conv.reshape(VOCAB, EMBED)
    idxp = token_ids.T.reshape(SB, 8, CB, TI).transpose(0, 2, 1, 3)
    o = _sc_gather(idxp, tab)
    return o.transpose(2, 4, 0, 1, 3).reshape(BATCH, SEQ, EMBED)


# speedup vs baseline: 1.3099x; 1.0097x over previous
import functools

import jax
import jax.numpy as jnp
from jax import lax
from jax.experimental import pallas as pl
from jax.experimental.pallas import tpu as pltpu
from jax.experimental.pallas import tpu_sc as plsc

VOCAB = 1000000
EMBED = 32
BATCH = 4096
SEQ = 200

_info = plsc.get_sparse_core_info()
NC = _info.num_cores
NS = _info.num_subcores
NW = NC * NS

SB = SEQ // 8
CB = BATCH // 128
FB = EMBED // 8
TI = 128
L = 16
SG = 4
NR = SEQ // SG


NFULL = VOCAB // TI
NTAIL = VOCAB - NFULL * TI


def _sc_convert(t_t, tail):
    mesh = plsc.VectorSubcoreMesh(core_axis_name="c", subcore_axis_name="s")

    GRP = 1
    CW = GRP * TI
    ORW = GRP * EMBED
    NGT = NFULL // GRP
    NRF = NGT // NW

    @functools.partial(
        pl.kernel,
        mesh=mesh,
        out_type=jax.ShapeDtypeStruct((VOCAB * EMBED // TI, TI), jnp.float32),
        scratch_types=[
            pltpu.VMEM((2, FB * GRP * 8, TI), jnp.float32),
            pltpu.VMEM((2, ORW, TI), jnp.float32),
            pltpu.VMEM((NTAIL * EMBED // TI, TI), jnp.float32),
            pltpu.SemaphoreType.DMA((2,)),
            pltpu.SemaphoreType.DMA((2,)),
            pltpu.SemaphoreType.DMA,
        ],
        compiler_params=pltpu.CompilerParams(
            use_tc_tiling_on_sc=True, needs_layout_passes=False),
    )
    def k(tt_hbm, tail_hbm, out_hbm, vbuf, obuf, tbuf, gsem, wsem, tsem):
        w = lax.axis_index("s") * NC + lax.axis_index("c")
        z = w * 0
        has_extra = w < NGT - NRF * NW

        def in_start(r, b):
            g = r * NW + w
            pltpu.async_copy(
                tt_hbm.at[:, pl.ds(g * TI, TI)], vbuf.at[b], gsem.at[b])

        def in_wait(r, b):
            g = r * NW + w
            pltpu.make_async_copy(
                tt_hbm.at[:, pl.ds(g * TI, TI)], vbuf.at[b], gsem.at[b]).wait()

        def out_start(r, b):
            g = r * NW + w
            pltpu.async_copy(
                obuf.at[b], out_hbm.at[pl.ds(g * ORW, ORW)], wsem.at[b])

        def out_wait(r, b):
            g = r * NW + w
            pltpu.make_async_copy(
                obuf.at[b], out_hbm.at[pl.ds(g * ORW, ORW)], wsem.at[b]).wait()

        def transpose_tiles(b):
            vb = vbuf.at[b]
            fl = lax.iota(jnp.int32, L)
            rowc = [[(((c00 + fl) // 8) * GRP + cb2) * 8 + (c00 + fl) % 8
                     for cb2 in range(GRP)] for c00 in (0, 16)]

            for cb2 in range(GRP):
                for r4 in range(EMBED):
                    tvs = [jnp.full((L,), r4 * 4 + q + z, jnp.int32)
                           for q in range(4)]
                    vals = [plsc.load_gather(
                        vb, [rowc[c0g % 2][cb2], tvs[c0g // 2]])
                        for c0g in range(TI // L)]
                    for c0g in range(TI // L):
                        obuf[b, cb2 * EMBED + r4, pl.ds(c0g * L, L)] = vals[c0g]

        in_start(0, 0)

        def body(g, carry):
            for b in range(2):
                r = g * 2 + b
                bp = 1 - b

                @pl.when(jnp.logical_or(r + 1 < NRF,
                                        jnp.logical_and(r + 1 == NRF, has_extra)))
                def _():
                    in_start(r + 1, bp)

                in_wait(r, b)

                @pl.when(r >= 2)
                def _():
                    out_wait(r - 2, b)

                transpose_tiles(b)
                out_start(r, b)
            return carry

        lax.fori_loop(0, NRF // 2, body, 0)

        out_wait(NRF - 2, 0)

        @pl.when(has_extra)
        def _():
            in_wait(NRF, 0)
            transpose_tiles(0)
            out_start(NRF, 0)
            out_wait(NRF, 0)

        out_wait(NRF - 1, 1)

        @pl.when(w == 0)
        def _():
            pltpu.async_copy(tail_hbm, tbuf, tsem).wait()
            pltpu.sync_copy(tbuf, out_hbm.at[pl.ds(NFULL * EMBED, NTAIL * EMBED // TI)])

    return k(t_t, tail)


def _sc_gather(idxp, table):
    mesh = plsc.VectorSubcoreMesh(core_axis_name="c", subcore_axis_name="s")

    @functools.partial(
        pl.kernel,
        mesh=mesh,
        out_type=jax.ShapeDtypeStruct((SEQ, FB, CB, 8, TI), jnp.float32),
        scratch_types=[
            pltpu.VMEM((2, SG * TI), jnp.int32),
            pltpu.VMEM((2, SG * TI, EMBED), jnp.float32),
            pltpu.VMEM((2, SG, FB, 8, TI), jnp.float32),
            pltpu.SemaphoreType.DMA((2,)),
            pltpu.SemaphoreType.DMA((2,)),
            pltpu.SemaphoreType.DMA((2,)),
        ],
        compiler_params=pltpu.CompilerParams(
            use_tc_tiling_on_sc=False, needs_layout_passes=False),
    )
    def k(idx_hbm, table_hbm, out_hbm, idx_v, rows_v, obuf, isem, gsem, wsem):
        w = lax.axis_index("s") * NC + lax.axis_index("c")
        z = w * 0

        def idx_load(r, b):
            pltpu.async_copy(idx_hbm.at[w, r], idx_v.at[b], isem.at[b])

        def idx_wait(r, b):
            pltpu.make_async_copy(
                idx_hbm.at[w, r], idx_v.at[b], isem.at[b]).wait()

        def gathers_start(b):
            pltpu.async_copy(
                table_hbm.at[idx_v.at[b]], rows_v.at[b], gsem.at[b])

        def gathers_wait(b):
            pltpu.make_async_copy(
                table_hbm.at[idx_v.at[b]], rows_v.at[b], gsem.at[b]).wait()

        def wb_start(r, b):
            pltpu.async_copy(
                obuf.at[b], out_hbm.at[pl.ds(r * SG, SG), :, w], wsem.at[b])

        def wb_wait(r, b):
            pltpu.make_async_copy(
                obuf.at[b], out_hbm.at[pl.ds(r * SG, SG), :, w],
                wsem.at[b]).wait()

        def transpose_round(b):
            rowv = [lax.iota(jnp.int32, L) + t0 * L for t0 in range(TI // L)]
            for si in range(SG):
                rows = rows_v.at[b, pl.ds(si * TI, TI)]
                for f in range(EMBED):
                    fv = jnp.full((L,), f + z, jnp.int32)
                    vals = [plsc.load_gather(rows, [rowv[t0], fv])
                            for t0 in range(TI // L)]
                    for t0 in range(TI // L):
                        obuf[b, si, f // 8, f % 8, pl.ds(t0 * L, L)] = vals[t0]

        idx_load(0, 0)
        idx_load(1, 1)

        def body(g, carry):
            for b in range(2):
                r = g * 2 + b
                bp = 1 - b
                idx_wait(r, b)
                gathers_start(b)

                @pl.when(r >= 1)
                def _():
                    gathers_wait(bp)

                    @pl.when(r + 1 < NR)
                    def _():
                        idx_load(r + 1, bp)

                    @pl.when(r >= 3)
                    def _():
                        wb_wait(r - 3, bp)

                    transpose_round(bp)
                    wb_start(r - 1, bp)
            return carry

        lax.fori_loop(0, NR // 2, body, 0)

        gathers_wait(1)
        wb_wait(NR - 3, 1)
        transpose_round(1)
        wb_start(NR - 1, 1)
        wb_wait(NR - 2, 0)
        wb_wait(NR - 1, 1)

    return k(idxp, table)


def kernel(token_ids, table):
    tail = table[NFULL * TI:, :].reshape(NTAIL * EMBED // TI, TI)
    conv = _sc_convert(table.T, tail)
    tab = conv.reshape(VOCAB, EMBED)
    idxp = (token_ids.T.reshape(NR, SG, CB, TI)
            .transpose(2, 0, 1, 3).reshape(CB, NR, SG * TI))
    o = _sc_gather(idxp, tab)
    return o.transpose(2, 4, 0, 1, 3).reshape(BATCH, SEQ, EMBED)


# speedup vs baseline: 1.5898x; 1.2137x over previous
import functools

import jax
import jax.numpy as jnp
from jax import lax
from jax.experimental import pallas as pl
from jax.experimental.pallas import tpu as pltpu
from jax.experimental.pallas import tpu_sc as plsc

VOCAB = 1000000
EMBED = 32
BATCH = 4096
SEQ = 200

_info = plsc.get_sparse_core_info()
NC = _info.num_cores
NS = _info.num_subcores
NW = NC * NS

SB = SEQ // 8
CB = BATCH // 128
FB = EMBED // 8
TI = 128
L = 16
SG = 4
NR = SEQ // SG


def _sc_gather(idxp, table):
    mesh = plsc.VectorSubcoreMesh(core_axis_name="c", subcore_axis_name="s")

    @functools.partial(
        pl.kernel,
        mesh=mesh,
        out_type=jax.ShapeDtypeStruct((SEQ, FB, CB, 8, TI), jnp.float32),
        scratch_types=[
            pltpu.VMEM((2, SG, TI), jnp.int32),
            pltpu.VMEM((2, SG, TI, EMBED), jnp.float32),
            pltpu.VMEM((2, SG, FB, 8, TI), jnp.float32),
            pltpu.SemaphoreType.DMA((2,)),
            pltpu.SemaphoreType.DMA((2,)),
            pltpu.SemaphoreType.DMA((2,)),
        ],
        compiler_params=pltpu.CompilerParams(
            use_tc_tiling_on_sc=False, needs_layout_passes=False),
    )
    def k(idx_hbm, table_hbm, out_hbm, idx_v, rows_v, obuf, isem, gsem, wsem):
        w = lax.axis_index("s") * NC + lax.axis_index("c")

        def idx_load(r, b):
            pltpu.async_copy(
                idx_hbm.at[r // 2, w, pl.ds((r % 2) * SG, SG)],
                idx_v.at[b], isem.at[b])

        def idx_wait(r, b):
            pltpu.make_async_copy(
                idx_hbm.at[r // 2, w, pl.ds((r % 2) * SG, SG)],
                idx_v.at[b], isem.at[b]).wait()

        def gathers_start(b):
            for si in range(SG):
                pltpu.async_copy(
                    table_hbm.at[idx_v.at[b, si]], rows_v.at[b, si],
                    gsem.at[b])

        def gathers_wait(b):
            for si in range(SG):
                pltpu.make_async_copy(
                    table_hbm.at[idx_v.at[b, si]], rows_v.at[b, si],
                    gsem.at[b]).wait()

        def wb_start(r, b):
            pltpu.async_copy(
                obuf.at[b], out_hbm.at[pl.ds(r * SG, SG), :, w], wsem.at[b])

        def wb_wait(r, b):
            pltpu.make_async_copy(
                obuf.at[b], out_hbm.at[pl.ds(r * SG, SG), :, w],
                wsem.at[b]).wait()

        def transpose_round(b):
            for si in range(SG):
                rows = rows_v.at[b, si]
                for t0 in range(TI // L):
                    row = lax.iota(jnp.int32, L) + (t0 * L)
                    vals = [
                        plsc.load_gather(rows, [row, jnp.full((L,), f, jnp.int32)])
                        for f in range(EMBED)
                    ]
                    for f in range(EMBED):
                        obuf[b, si, f // 8, f % 8, pl.ds(t0 * L, L)] = vals[f]

        idx_load(0, 0)
        idx_load(1, 1)

        def body(g, carry):
            for b in range(2):
                r = g * 2 + b
                bp = 1 - b
                idx_wait(r, b)
                gathers_start(b)

                @pl.when(r >= 1)
                def _():
                    gathers_wait(bp)

                    @pl.when(r + 1 < NR)
                    def _():
                        idx_load(r + 1, bp)

                    @pl.when(r >= 3)
                    def _():
                        wb_wait(r - 3, bp)

                    transpose_round(bp)
                    wb_start(r - 1, bp)
            return carry

        lax.fori_loop(0, NR // 2, body, 0)

        gathers_wait(1)
        wb_wait(NR - 3, 1)
        transpose_round(1)
        wb_start(NR - 1, 1)
        wb_wait(NR - 2, 0)
        wb_wait(NR - 1, 1)

    return k(idxp, table)


def kernel(token_ids, table):
    idxp = token_ids.T.reshape(SB, 8, CB, TI).transpose(0, 2, 1, 3)
    o = _sc_gather(idxp, table)
    return o.transpose(2, 4, 0, 1, 3).reshape(BATCH, SEQ, EMBED)
